# Initial kernel scaffold; baseline (speedup 1.0000x reference)
import jax
import jax.numpy as jnp
from jax.experimental import pallas as pl


def kernel(sparse_idx, seq_idx, seq_lengths, dense, sparse_tables, seq_table):
    raise NotImplementedError("write your pallas kernel here")


# speedup vs baseline: 3.2911x; 3.2911x over previous
import functools

import jax
import jax.numpy as jnp
from jax import lax
from jax.experimental import pallas as pl
from jax.experimental.pallas import tpu as pltpu
from jax.experimental.pallas import tpu_sc as plsc

_NC = 2
_NS = 16


@functools.partial(jax.jit, static_argnums=(0, 1, 2, 3))
def _sc_input_layer(F, V, D, L, tabt, sidxt, seqt, qidx3, lens):
    B = lens.shape[0]
    NW = _NC * _NS
    NBB = 16
    BBL = B // NBB
    NV = B // 16

    mesh = plsc.VectorSubcoreMesh(core_axis_name="c", subcore_axis_name="s")

    @functools.partial(
        pl.kernel,
        out_type=(
            jax.ShapeDtypeStruct((F * D * (B // 128), 128), jnp.float32),
            jax.ShapeDtypeStruct((D * (B // 128), 128), jnp.float32),
        ),
        mesh=mesh,
        compiler_params=pltpu.CompilerParams(
            use_tc_tiling_on_sc=True, needs_layout_passes=False),
        scratch_types=[
            pltpu.VMEM((V,), jnp.float32),
            pltpu.VMEM((B,), jnp.int32),
            pltpu.VMEM((L, BBL), jnp.int32),
            pltpu.VMEM((B,), jnp.float32),
            pltpu.VMEM((B // 128, 128), jnp.float32),
            pltpu.VMEM((B // 128, 128), jnp.float32),
        ],
    )
    def k(tabt_h, sidxt_h, seqt_h, qidx3_h, lens_h, sout_h, pout_h,
          row_v, sidx_v, qidx_v, lens_v, res_v, pres_v):
        d = lax.axis_index("s") * _NC + lax.axis_index("c")

        pltpu.sync_copy(lens_h, lens_v)
        pltpu.sync_copy(seqt_h.at[d, :], row_v)

        def qblock(bb, carry):
            pltpu.sync_copy(qidx3_h.at[bb], qidx_v)

            def qvec(bv, c2):
                lenv = lens_v[pl.ds(bb * BBL + bv * 16, 16)]
                acc = jnp.zeros((16,), jnp.float32)
                for j in range(L):
                    iv = qidx_v[j, pl.ds(bv * 16, 16)]
                    g = plsc.load_gather(row_v, [iv])
                    m = jnp.minimum(jnp.maximum(lenv - float(j), 0.0), 1.0)
                    acc = acc + g * m
                rv = 1.0 / jnp.maximum(lenv, 1.0)
                b0 = bb * BBL + bv * 16
                pres_v[b0 // 128, pl.ds(b0 % 128, 16)] = acc * rv
                return c2

            lax.fori_loop(0, BBL // 16, qvec, 0)
            return carry

        lax.fori_loop(0, NBB, qblock, 0)
        pltpu.sync_copy(pres_v, pout_h.at[pl.ds(d * (B // 128), B // 128)])

        def fbody(f, carry):
            pltpu.sync_copy(tabt_h.at[f, d, :], row_v)
            pltpu.sync_copy(sidxt_h.at[f], sidx_v)

            def svec(bv, c2):
                iv = sidx_v[pl.ds(bv * 16, 16)]
                g = plsc.load_gather(row_v, [iv])
                res_v[bv // 8, pl.ds((bv % 8) * 16, 16)] = g
                return c2

            lax.fori_loop(0, NV, svec, 0)
            fd = f * D + d
            pltpu.sync_copy(res_v, sout_h.at[pl.ds(fd * (B // 128), B // 128)])
            return carry

        lax.fori_loop(0, F, fbody, 0)

    return k(tabt, sidxt, seqt, qidx3, lens)


def kernel(sparse_idx, seq_idx, seq_lengths, dense, sparse_tables, seq_table):
    B, F = sparse_idx.shape
    L = seq_idx.shape[1]
    V, D = seq_table.shape
    tabt = sparse_tables.transpose(0, 2, 1)
    seqt = seq_table.T
    sidxt = sparse_idx.astype(jnp.int32).T
    qidx3 = (seq_idx.astype(jnp.int32).T
             .reshape(L, 16, B // 16).transpose(1, 0, 2))
    lens = seq_lengths.astype(jnp.float32)
    souto, pouto = _sc_input_layer(F, V, D, L, tabt, sidxt, seqt, qidx3, lens)
    sparse_out = souto.reshape(F * D, B).T
    pooled = pouto.reshape(D, B).T
    return jnp.concatenate([sparse_out, pooled, dense], axis=-1)
